# baseline (speedup 1.0000x reference)
import functools

import jax
import jax.numpy as jnp
from jax import lax
from jax.experimental import pallas as pl
from jax.experimental.pallas import tpu as pltpu
from jax.experimental.pallas import tpu_sc as plsc

_B, _H, _W = 32, 512, 512
_NC, _NS, _NW = 2, 16, 32

_SC_BATCHES = 8
_CHUNK_ROWS = 16

_LOG1P = (1.4158018e-04, 9.9542665e-01, -4.6407071e-01, 2.1640858e-01,
          -5.4862313e-02)


def _log1p_exp_neg_abs(x):
    u = jnp.exp(-jnp.abs(x))
    p = jnp.full_like(u, _LOG1P[4])
    for c in _LOG1P[3::-1]:
        p = p * u + c
    return p


def _sc_reduce(xw, xs, pm, nm, k_sc):
    spb = _NW // k_sc
    rows_per_sub = _H // spb
    n_chunks = rows_per_sub // _CHUNK_ROWS
    mesh = plsc.VectorSubcoreMesh(core_axis_name="c", subcore_axis_name="s")

    @functools.partial(
        pl.kernel, mesh=mesh,
        out_type=jax.ShapeDtypeStruct((_NW, 3, 16), jnp.float32),
        scratch_types=[pltpu.VMEM((_CHUNK_ROWS, _W), jnp.float32)] * 4
        + [pltpu.VMEM((3, 16), jnp.float32)],
    )
    def sc_kernel(xw_h, xs_h, pm_h, nm_h, out_h, xw_v, xs_v, pm_v, nm_v,
                  part_v):
        wid = lax.axis_index("c") * _NS + lax.axis_index("s")
        b = wid // spb
        r0 = (wid % spb) * rows_per_sub

        def chunk_body(ci, accs):
            row = r0 + ci * _CHUNK_ROWS
            pltpu.sync_copy(xw_h.at[b, 0, pl.ds(row, _CHUNK_ROWS), :], xw_v)
            pltpu.sync_copy(xs_h.at[b, 0, pl.ds(row, _CHUNK_ROWS), :], xs_v)
            pltpu.sync_copy(pm_h.at[b, 0, pl.ds(row, _CHUNK_ROWS), :], pm_v)
            pltpu.sync_copy(nm_h.at[b, 0, pl.ds(row, _CHUNK_ROWS), :], nm_v)

            def col_body(j, accs2):
                a_t, a_x, a_c = accs2
                sl = pl.ds(j * 16, 16)
                for r in range(_CHUNK_ROWS):
                    xw_l = xw_v[r, sl]
                    xs_l = xs_v[r, sl]
                    m = (pm_v[r, sl] > 0.5) & (nm_v[r, sl] > 0.5)
                    t = (jnp.maximum(xw_l, 0.0) + _log1p_exp_neg_abs(xw_l)
                         + jnp.maximum(xs_l, 0.0) + _log1p_exp_neg_abs(xs_l)
                         - jnp.where((xw_l > 0.6) | (xw_l < 0.4),
                                     xw_l * xs_l, 0.0))
                    a_t = a_t + jnp.where(m, t, 0.0)
                    a_x = a_x + jnp.where(m, xw_l, 0.0)
                    a_c = a_c + jnp.where(m, 1.0, 0.0)
                return (a_t, a_x, a_c)

            return lax.fori_loop(0, _W // 16, col_body, accs)

        z = jnp.zeros((16,), jnp.float32)
        a_t, a_x, a_c = lax.fori_loop(0, n_chunks, chunk_body, (z, z, z))

        part_v[0, :] = a_t
        part_v[1, :] = a_x
        part_v[2, :] = a_c
        pltpu.sync_copy(part_v, out_h.at[wid])

    return sc_kernel(xw, xs, pm, nm)


def _tc_kernel_body(xw_ref, xs_ref, pm_ref, nm_ref, num_ref, cnt_ref,
                    sxw_ref):
    i = pl.program_id(0)

    @pl.when(i == 0)
    def _init():
        num_ref[:, :] = jnp.zeros((1, 1), jnp.float32)
        cnt_ref[:, :] = jnp.zeros((1, 1), jnp.float32)

    xw = xw_ref[0]
    xs = xs_ref[0]
    mask = (pm_ref[0] > 0.5) & (nm_ref[0] > 0.5)

    sp_w = jnp.maximum(-xw, 0.0) + jnp.log1p(jnp.exp(-jnp.abs(xw)))
    sp_s = jnp.maximum(-xs, 0.0) + jnp.log1p(jnp.exp(-jnp.abs(xs)))

    pseudo = jnp.where((xw > 0.6) | (xw < 0.4), xw, 0.0)
    t_sum = (sp_w + sp_s) + (xw + xs) - pseudo * xs

    num_ref[:, :] += jnp.sum(jnp.where(mask, t_sum, 0.0)).reshape(1, 1)
    cnt_ref[:, :] += jnp.sum(jnp.where(mask, 1.0, 0.0)).reshape(1, 1)
    sxw_ref[:, :, :] = jnp.sum(jnp.where(mask, xw, 0.0)).reshape(1, 1, 1)


def _tc_reduce(xw, xs, pm, nm, b0):
    nb = _B - b0
    blk = pl.BlockSpec((1, _H, _W), lambda i: (i + b0, 0, 0))
    scal_blk = pl.BlockSpec((1, 1), lambda i: (0, 0))
    return pl.pallas_call(
        _tc_kernel_body,
        grid=(nb,),
        in_specs=[blk, blk, blk, blk],
        out_specs=[scal_blk, scal_blk,
                   pl.BlockSpec((1, 1, 1), lambda i: (i, 0, 0))],
        out_shape=[jax.ShapeDtypeStruct((1, 1), jnp.float32),
                   jax.ShapeDtypeStruct((1, 1), jnp.float32),
                   jax.ShapeDtypeStruct((nb, 1, 1), jnp.float32)],
    )(xw, xs, pm, nm)


def kernel(logits_w, logits_s, prostate_mask, needle_mask, ood_mask,
           label, involvement):
    del ood_mask, involvement
    labf = label.astype(jnp.float32)
    k_sc = _SC_BATCHES
    spb = _NW // k_sc if k_sc else 1

    num = jnp.float32(0.0)
    cnt = jnp.float32(0.0)
    lab_dot = jnp.float32(0.0)

    if k_sc:
        parts = _sc_reduce(logits_w, logits_s, prostate_mask, needle_mask,
                           k_sc)
        p = parts.reshape(k_sc, spb, 3, 16)
        num = num + jnp.sum(p[:, :, 0, :])
        cnt = cnt + jnp.sum(p[:, :, 2, :])
        sxw_sc = jnp.sum(p[:, :, 1, :], axis=(1, 2))
        lab_dot = lab_dot + jnp.dot(labf[:k_sc], sxw_sc)

    if k_sc < _B:
        xw = logits_w.reshape(_B, _H, _W)
        xs = logits_s.reshape(_B, _H, _W)
        pm = prostate_mask.reshape(_B, _H, _W)
        nm = needle_mask.reshape(_B, _H, _W)
        num_tc, cnt_tc, sxw_tc = _tc_reduce(xw, xs, pm, nm, k_sc)
        num = num + num_tc[0, 0]
        cnt = cnt + cnt_tc[0, 0]
        lab_dot = lab_dot + jnp.dot(labf[k_sc:], sxw_tc.reshape(_B - k_sc))

    return (0.5 * (num - lab_dot) / cnt).astype(jnp.float32)


# speedup vs baseline: 1.4598x; 1.4598x over previous
import functools

import jax
import jax.numpy as jnp
from jax import lax
from jax.experimental import pallas as pl
from jax.experimental.pallas import tpu as pltpu
from jax.experimental.pallas import tpu_sc as plsc

_B, _H, _W = 32, 512, 512
_NC, _NS, _NW = 2, 16, 32

_SC_BATCHES = 4
_CHUNK_ROWS = 16

_LOG1P = (5.6293300e-07, 9.9995744e-01, -4.9920639e-01, 3.2697237e-01,
          -2.2283472e-01, 1.3076335e-01, -5.2623954e-02, 1.0118902e-02)


def _softplus_neg_sc(x):
    u = jnp.exp(-jnp.abs(x))
    p = jnp.full_like(u, _LOG1P[7])
    for c in _LOG1P[6::-1]:
        p = p * u + c
    return jnp.maximum(-x, 0.0) + p


def _sc_reduce(xw, xs, pm, nm, k_sc):
    spb = _NW // k_sc
    rows_per_sub = _H // spb
    n_chunks = rows_per_sub // _CHUNK_ROWS
    mesh = plsc.VectorSubcoreMesh(core_axis_name="c", subcore_axis_name="s")

    @functools.partial(
        pl.kernel, mesh=mesh,
        out_type=jax.ShapeDtypeStruct((_NW, 3, 16), jnp.float32),
        scratch_types=[pltpu.VMEM((_CHUNK_ROWS, _W), jnp.float32)] * 4
        + [pltpu.VMEM((3, 16), jnp.float32)],
    )
    def sc_kernel(xw_h, xs_h, pm_h, nm_h, out_h, xw_v, xs_v, pm_v, nm_v,
                  part_v):
        wid = lax.axis_index("c") * _NS + lax.axis_index("s")
        b = wid // spb
        r0 = (wid % spb) * rows_per_sub

        def chunk_body(ci, accs):
            row = r0 + ci * _CHUNK_ROWS
            pltpu.sync_copy(xw_h.at[b, 0, pl.ds(row, _CHUNK_ROWS), :], xw_v)
            pltpu.sync_copy(xs_h.at[b, 0, pl.ds(row, _CHUNK_ROWS), :], xs_v)
            pltpu.sync_copy(pm_h.at[b, 0, pl.ds(row, _CHUNK_ROWS), :], pm_v)
            pltpu.sync_copy(nm_h.at[b, 0, pl.ds(row, _CHUNK_ROWS), :], nm_v)

            def col_body(j, accs2):
                a_t, a_x, a_c = accs2
                sl = pl.ds(j * 16, 16)
                for r in range(_CHUNK_ROWS):
                    xw_l = xw_v[r, sl]
                    xs_l = xs_v[r, sl]
                    m = (pm_v[r, sl] > 0.5) & (nm_v[r, sl] > 0.5)
                    sp = _softplus_neg_sc(xw_l) + _softplus_neg_sc(xs_l)
                    ps_xs = jnp.where((xw_l > 0.6) | (xw_l < 0.4),
                                      xw_l * xs_l, 0.0)
                    t = sp + xw_l + xs_l - ps_xs
                    a_t = a_t + jnp.where(m, t, 0.0)
                    a_x = a_x + jnp.where(m, xw_l, 0.0)
                    a_c = a_c + jnp.where(m, 1.0, 0.0)
                return (a_t, a_x, a_c)

            return lax.fori_loop(0, _W // 16, col_body, accs)

        z = jnp.zeros((16,), jnp.float32)
        a_t, a_x, a_c = lax.fori_loop(0, n_chunks, chunk_body, (z, z, z))

        part_v[0, :] = a_t
        part_v[1, :] = a_x
        part_v[2, :] = a_c
        pltpu.sync_copy(part_v, out_h.at[wid])

    return sc_kernel(xw, xs, pm, nm)


def _tc_kernel_body(xw_ref, xs_ref, pm_ref, nm_ref, num_ref, cnt_ref,
                    sxw_ref):
    i = pl.program_id(0)

    @pl.when(i == 0)
    def _init():
        num_ref[:, :] = jnp.zeros((1, 1), jnp.float32)
        cnt_ref[:, :] = jnp.zeros((1, 1), jnp.float32)

    xw = xw_ref[0]
    xs = xs_ref[0]
    mask = (pm_ref[0] > 0.5) & (nm_ref[0] > 0.5)

    sp_w = jnp.maximum(-xw, 0.0) + jnp.log1p(jnp.exp(-jnp.abs(xw)))
    sp_s = jnp.maximum(-xs, 0.0) + jnp.log1p(jnp.exp(-jnp.abs(xs)))

    pseudo = jnp.where((xw > 0.6) | (xw < 0.4), xw, 0.0)
    t_sum = (sp_w + sp_s) + (xw + xs) - pseudo * xs

    num_ref[:, :] += jnp.sum(jnp.where(mask, t_sum, 0.0)).reshape(1, 1)
    cnt_ref[:, :] += jnp.sum(jnp.where(mask, 1.0, 0.0)).reshape(1, 1)
    sxw_ref[:, :, :] = jnp.sum(jnp.where(mask, xw, 0.0)).reshape(1, 1, 1)


def _tc_reduce(xw, xs, pm, nm, b0):
    nb = _B - b0
    blk = pl.BlockSpec((1, _H, _W), lambda i: (i + b0, 0, 0))
    scal_blk = pl.BlockSpec((1, 1), lambda i: (0, 0))
    return pl.pallas_call(
        _tc_kernel_body,
        grid=(nb,),
        in_specs=[blk, blk, blk, blk],
        out_specs=[scal_blk, scal_blk,
                   pl.BlockSpec((1, 1, 1), lambda i: (i, 0, 0))],
        out_shape=[jax.ShapeDtypeStruct((1, 1), jnp.float32),
                   jax.ShapeDtypeStruct((1, 1), jnp.float32),
                   jax.ShapeDtypeStruct((nb, 1, 1), jnp.float32)],
    )(xw, xs, pm, nm)


def kernel(logits_w, logits_s, prostate_mask, needle_mask, ood_mask,
           label, involvement):
    del ood_mask, involvement
    labf = label.astype(jnp.float32)
    k_sc = _SC_BATCHES
    spb = _NW // k_sc if k_sc else 1

    num = jnp.float32(0.0)
    cnt = jnp.float32(0.0)
    lab_dot = jnp.float32(0.0)

    if k_sc:
        parts = _sc_reduce(logits_w, logits_s, prostate_mask, needle_mask,
                           k_sc)
        p = parts.reshape(k_sc, spb, 3, 16)
        num = num + jnp.sum(p[:, :, 0, :])
        cnt = cnt + jnp.sum(p[:, :, 2, :])
        sxw_sc = jnp.sum(p[:, :, 1, :], axis=(1, 2))
        lab_dot = lab_dot + jnp.dot(labf[:k_sc], sxw_sc)

    if k_sc < _B:
        xw = logits_w.reshape(_B, _H, _W)
        xs = logits_s.reshape(_B, _H, _W)
        pm = prostate_mask.reshape(_B, _H, _W)
        nm = needle_mask.reshape(_B, _H, _W)
        num_tc, cnt_tc, sxw_tc = _tc_reduce(xw, xs, pm, nm, k_sc)
        num = num + num_tc[0, 0]
        cnt = cnt + cnt_tc[0, 0]
        lab_dot = lab_dot + jnp.dot(labf[k_sc:], sxw_tc.reshape(_B - k_sc))

    return (0.5 * (num - lab_dot) / cnt).astype(jnp.float32)


# speedup vs baseline: 1.4605x; 1.0004x over previous
import functools

import jax
import jax.numpy as jnp
from jax import lax
from jax.experimental import pallas as pl
from jax.experimental.pallas import tpu as pltpu
from jax.experimental.pallas import tpu_sc as plsc

_B, _H, _W = 32, 512, 512
_NC, _NS, _NW = 2, 16, 32

_SC_BATCHES = 4
_CHUNK_ROWS = 16

_LOG1P = (5.6293300e-07, 9.9995744e-01, -4.9920639e-01, 3.2697237e-01,
          -2.2283472e-01, 1.3076335e-01, -5.2623954e-02, 1.0118902e-02)


def _softplus_neg_sc(x):
    u = jnp.exp(-jnp.abs(x))
    p = jnp.full_like(u, _LOG1P[7])
    for c in _LOG1P[6::-1]:
        p = p * u + c
    return jnp.maximum(-x, 0.0) + p


def _sc_reduce(xw, xs, pm, nm, k_sc):
    spb = _NW // k_sc
    rows_per_sub = _H // spb
    n_chunks = rows_per_sub // _CHUNK_ROWS
    mesh = plsc.VectorSubcoreMesh(core_axis_name="c", subcore_axis_name="s")

    @functools.partial(
        pl.kernel, mesh=mesh,
        out_type=jax.ShapeDtypeStruct((_NW, 3, 16), jnp.float32),
        scratch_types=[pltpu.VMEM((_CHUNK_ROWS, _W), jnp.float32)] * 4
        + [pltpu.VMEM((3, 16), jnp.float32), pltpu.SemaphoreType.DMA],
    )
    def sc_kernel(xw_h, xs_h, pm_h, nm_h, out_h, xw_v, xs_v, pm_v, nm_v,
                  part_v, sem):
        wid = lax.axis_index("c") * _NS + lax.axis_index("s")
        b = wid // spb
        r0 = (wid % spb) * rows_per_sub

        def chunk_body(ci, accs):
            row = r0 + ci * _CHUNK_ROWS
            sl_h = pl.ds(row, _CHUNK_ROWS)
            copies = [
                pltpu.make_async_copy(s.at[b, 0, sl_h, :], d, sem)
                for s, d in ((xw_h, xw_v), (xs_h, xs_v),
                             (pm_h, pm_v), (nm_h, nm_v))]
            for cp in copies:
                cp.start()
            for cp in copies:
                cp.wait()

            def col_body(j, accs2):
                a_t, a_x, a_c = accs2
                sl = pl.ds(j * 16, 16)
                for r in range(_CHUNK_ROWS):
                    xw_l = xw_v[r, sl]
                    xs_l = xs_v[r, sl]
                    m = (pm_v[r, sl] > 0.5) & (nm_v[r, sl] > 0.5)
                    sp = _softplus_neg_sc(xw_l) + _softplus_neg_sc(xs_l)
                    ps_xs = jnp.where((xw_l > 0.6) | (xw_l < 0.4),
                                      xw_l * xs_l, 0.0)
                    t = sp + xw_l + xs_l - ps_xs
                    a_t = a_t + jnp.where(m, t, 0.0)
                    a_x = a_x + jnp.where(m, xw_l, 0.0)
                    a_c = a_c + jnp.where(m, 1.0, 0.0)
                return (a_t, a_x, a_c)

            return lax.fori_loop(0, _W // 16, col_body, accs)

        z = jnp.zeros((16,), jnp.float32)
        a_t, a_x, a_c = lax.fori_loop(0, n_chunks, chunk_body, (z, z, z))

        part_v[0, :] = a_t
        part_v[1, :] = a_x
        part_v[2, :] = a_c
        pltpu.sync_copy(part_v, out_h.at[wid])

    return sc_kernel(xw, xs, pm, nm)


def _tc_kernel_body(xw_ref, xs_ref, pm_ref, nm_ref, num_ref, cnt_ref,
                    sxw_ref):
    i = pl.program_id(0)

    @pl.when(i == 0)
    def _init():
        num_ref[:, :] = jnp.zeros((1, 1), jnp.float32)
        cnt_ref[:, :] = jnp.zeros((1, 1), jnp.float32)

    xw = xw_ref[0]
    xs = xs_ref[0]
    mask = (pm_ref[0] > 0.5) & (nm_ref[0] > 0.5)

    sp_w = jnp.maximum(-xw, 0.0) + jnp.log1p(jnp.exp(-jnp.abs(xw)))
    sp_s = jnp.maximum(-xs, 0.0) + jnp.log1p(jnp.exp(-jnp.abs(xs)))

    pseudo = jnp.where((xw > 0.6) | (xw < 0.4), xw, 0.0)
    t_sum = (sp_w + sp_s) + (xw + xs) - pseudo * xs

    num_ref[:, :] += jnp.sum(jnp.where(mask, t_sum, 0.0)).reshape(1, 1)
    cnt_ref[:, :] += jnp.sum(jnp.where(mask, 1.0, 0.0)).reshape(1, 1)
    sxw_ref[:, :, :] = jnp.sum(jnp.where(mask, xw, 0.0)).reshape(1, 1, 1)


def _tc_reduce(xw, xs, pm, nm, b0):
    nb = _B - b0
    blk = pl.BlockSpec((1, _H, _W), lambda i: (i + b0, 0, 0))
    scal_blk = pl.BlockSpec((1, 1), lambda i: (0, 0))
    return pl.pallas_call(
        _tc_kernel_body,
        grid=(nb,),
        in_specs=[blk, blk, blk, blk],
        out_specs=[scal_blk, scal_blk,
                   pl.BlockSpec((1, 1, 1), lambda i: (i, 0, 0))],
        out_shape=[jax.ShapeDtypeStruct((1, 1), jnp.float32),
                   jax.ShapeDtypeStruct((1, 1), jnp.float32),
                   jax.ShapeDtypeStruct((nb, 1, 1), jnp.float32)],
    )(xw, xs, pm, nm)


def kernel(logits_w, logits_s, prostate_mask, needle_mask, ood_mask,
           label, involvement):
    del ood_mask, involvement
    labf = label.astype(jnp.float32)
    k_sc = _SC_BATCHES
    spb = _NW // k_sc if k_sc else 1

    num = jnp.float32(0.0)
    cnt = jnp.float32(0.0)
    lab_dot = jnp.float32(0.0)

    if k_sc:
        parts = _sc_reduce(logits_w, logits_s, prostate_mask, needle_mask,
                           k_sc)
        p = parts.reshape(k_sc, spb, 3, 16)
        num = num + jnp.sum(p[:, :, 0, :])
        cnt = cnt + jnp.sum(p[:, :, 2, :])
        sxw_sc = jnp.sum(p[:, :, 1, :], axis=(1, 2))
        lab_dot = lab_dot + jnp.dot(labf[:k_sc], sxw_sc)

    if k_sc < _B:
        xw = logits_w.reshape(_B, _H, _W)
        xs = logits_s.reshape(_B, _H, _W)
        pm = prostate_mask.reshape(_B, _H, _W)
        nm = needle_mask.reshape(_B, _H, _W)
        num_tc, cnt_tc, sxw_tc = _tc_reduce(xw, xs, pm, nm, k_sc)
        num = num + num_tc[0, 0]
        cnt = cnt + cnt_tc[0, 0]
        lab_dot = lab_dot + jnp.dot(labf[k_sc:], sxw_tc.reshape(_B - k_sc))

    return (0.5 * (num - lab_dot) / cnt).astype(jnp.float32)


# speedup vs baseline: 1.5987x; 1.0947x over previous
import functools

import jax
import jax.numpy as jnp
from jax import lax
from jax.experimental import pallas as pl
from jax.experimental.pallas import tpu as pltpu
from jax.experimental.pallas import tpu_sc as plsc

_B, _H, _W = 32, 512, 512
_NC, _NS, _NW = 2, 16, 32

_SC_BATCHES = 8
_CHUNK_ROWS = 16

_LOG1P = (5.6293300e-07, 9.9995744e-01, -4.9920639e-01, 3.2697237e-01,
          -2.2283472e-01, 1.3076335e-01, -5.2623954e-02, 1.0118902e-02)


def _softplus_neg_sc(x):
    u = jnp.exp(-jnp.abs(x))
    p = jnp.full_like(u, _LOG1P[7])
    for c in _LOG1P[6::-1]:
        p = p * u + c
    return jnp.maximum(-x, 0.0) + p


def _sc_reduce(xw, xs, pm, nm, k_sc):
    spb = _NW // k_sc
    rows_per_sub = _H // spb
    n_chunks = rows_per_sub // _CHUNK_ROWS
    mesh = plsc.VectorSubcoreMesh(core_axis_name="c", subcore_axis_name="s")

    @functools.partial(
        pl.kernel, mesh=mesh,
        out_type=jax.ShapeDtypeStruct((_NW, 3, 16), jnp.float32),
        scratch_types=[pltpu.VMEM((_CHUNK_ROWS, _W), jnp.float32)] * 4
        + [pltpu.VMEM((3, 16), jnp.float32), pltpu.SemaphoreType.DMA],
    )
    def sc_kernel(xw_h, xs_h, pm_h, nm_h, out_h, xw_v, xs_v, pm_v, nm_v,
                  part_v, sem):
        wid = lax.axis_index("c") * _NS + lax.axis_index("s")
        b = wid // spb
        r0 = (wid % spb) * rows_per_sub

        def chunk_body(ci, accs):
            row = r0 + ci * _CHUNK_ROWS
            sl_h = pl.ds(row, _CHUNK_ROWS)
            copies = [
                pltpu.make_async_copy(s.at[b, 0, sl_h, :], d, sem)
                for s, d in ((xw_h, xw_v), (xs_h, xs_v),
                             (pm_h, pm_v), (nm_h, nm_v))]
            for cp in copies:
                cp.start()
            for cp in copies:
                cp.wait()

            def col_body(j, accs2):
                a_t, a_x, a_c = accs2
                sl = pl.ds(j * 16, 16)
                for r in range(_CHUNK_ROWS):
                    xw_l = xw_v[r, sl]
                    xs_l = xs_v[r, sl]
                    m = (pm_v[r, sl] > 0.5) & (nm_v[r, sl] > 0.5)
                    sp = _softplus_neg_sc(xw_l) + _softplus_neg_sc(xs_l)
                    ps_xs = jnp.where((xw_l > 0.6) | (xw_l < 0.4),
                                      xw_l * xs_l, 0.0)
                    t = sp + xw_l + xs_l - ps_xs
                    a_t = a_t + jnp.where(m, t, 0.0)
                    a_x = a_x + jnp.where(m, xw_l, 0.0)
                    a_c = a_c + jnp.where(m, 1.0, 0.0)
                return (a_t, a_x, a_c)

            return lax.fori_loop(0, _W // 16, col_body, accs)

        z = jnp.zeros((16,), jnp.float32)
        a_t, a_x, a_c = lax.fori_loop(0, n_chunks, chunk_body, (z, z, z))

        part_v[0, :] = a_t
        part_v[1, :] = a_x
        part_v[2, :] = a_c
        pltpu.sync_copy(part_v, out_h.at[wid])

    return sc_kernel(xw, xs, pm, nm)


def _tc_kernel_body(xw_ref, xs_ref, pm_ref, nm_ref, num_ref, cnt_ref,
                    sxw_ref):
    i = pl.program_id(0)

    @pl.when(i == 0)
    def _init():
        num_ref[:, :] = jnp.zeros((1, 1), jnp.float32)
        cnt_ref[:, :] = jnp.zeros((1, 1), jnp.float32)

    xw = xw_ref[0]
    xs = xs_ref[0]
    mask = (pm_ref[0] > 0.5) & (nm_ref[0] > 0.5)

    sp_w = jnp.maximum(-xw, 0.0) + jnp.log1p(jnp.exp(-jnp.abs(xw)))
    sp_s = jnp.maximum(-xs, 0.0) + jnp.log1p(jnp.exp(-jnp.abs(xs)))

    pseudo = jnp.where((xw > 0.6) | (xw < 0.4), xw, 0.0)
    t_sum = (sp_w + sp_s) + (xw + xs) - pseudo * xs

    num_ref[:, :] += jnp.sum(jnp.where(mask, t_sum, 0.0)).reshape(1, 1)
    cnt_ref[:, :] += jnp.sum(jnp.where(mask, 1.0, 0.0)).reshape(1, 1)
    sxw_ref[:, :, :] = jnp.sum(jnp.where(mask, xw, 0.0)).reshape(1, 1, 1)


def _tc_reduce(xw, xs, pm, nm, b0):
    nb = _B - b0
    blk = pl.BlockSpec((1, _H, _W), lambda i: (i + b0, 0, 0))
    scal_blk = pl.BlockSpec((1, 1), lambda i: (0, 0))
    return pl.pallas_call(
        _tc_kernel_body,
        grid=(nb,),
        in_specs=[blk, blk, blk, blk],
        out_specs=[scal_blk, scal_blk,
                   pl.BlockSpec((1, 1, 1), lambda i: (i, 0, 0))],
        out_shape=[jax.ShapeDtypeStruct((1, 1), jnp.float32),
                   jax.ShapeDtypeStruct((1, 1), jnp.float32),
                   jax.ShapeDtypeStruct((nb, 1, 1), jnp.float32)],
    )(xw, xs, pm, nm)


def kernel(logits_w, logits_s, prostate_mask, needle_mask, ood_mask,
           label, involvement):
    del ood_mask, involvement
    labf = label.astype(jnp.float32)
    k_sc = _SC_BATCHES
    spb = _NW // k_sc if k_sc else 1

    num = jnp.float32(0.0)
    cnt = jnp.float32(0.0)
    lab_dot = jnp.float32(0.0)

    if k_sc:
        parts = _sc_reduce(logits_w, logits_s, prostate_mask, needle_mask,
                           k_sc)
        p = parts.reshape(k_sc, spb, 3, 16)
        num = num + jnp.sum(p[:, :, 0, :])
        cnt = cnt + jnp.sum(p[:, :, 2, :])
        sxw_sc = jnp.sum(p[:, :, 1, :], axis=(1, 2))
        lab_dot = lab_dot + jnp.dot(labf[:k_sc], sxw_sc)

    if k_sc < _B:
        xw = logits_w.reshape(_B, _H, _W)
        xs = logits_s.reshape(_B, _H, _W)
        pm = prostate_mask.reshape(_B, _H, _W)
        nm = needle_mask.reshape(_B, _H, _W)
        num_tc, cnt_tc, sxw_tc = _tc_reduce(xw, xs, pm, nm, k_sc)
        num = num + num_tc[0, 0]
        cnt = cnt + cnt_tc[0, 0]
        lab_dot = lab_dot + jnp.dot(labf[k_sc:], sxw_tc.reshape(_B - k_sc))

    return (0.5 * (num - lab_dot) / cnt).astype(jnp.float32)
